# baseline (speedup 1.0000x reference)
import functools

import jax
import jax.numpy as jnp
from jax import lax
from jax.experimental import pallas as pl
from jax.experimental.pallas import tpu as pltpu
from jax.experimental.pallas import tpu_sc as plsc

NUM_SELECT = 300
B, Q, C = 64, 900, 91
N = Q * C
NPAD = N + 4
NV = NPAD // 16
KSEL = 316
CAND = 384
CBUF = CAND + 16
BOXPAD = Q * 4 + 16
NW = 32
ROWS_PER_W = B // NW


def _sc_body(logits_hbm, cval_hbm, cidx_hbm,
             row_v, hist_v, cval_v, cidx_v):
    nc = 2
    wid = lax.axis_index("s") * nc + lax.axis_index("c")
    lane = lax.iota(jnp.int32, 16)
    lane_u = lane.astype(jnp.uint32)
    ones16 = jnp.ones((16,), jnp.int32)
    zeros16 = jnp.zeros((16,), jnp.int32)

    def key_of(i):
        x = row_v[pl.ds(i * 16, 16)]
        u = lax.bitcast_convert_type(x, jnp.uint32)
        return u ^ (jnp.uint32(0x80000000) | (jnp.uint32(0) - (u >> jnp.uint32(31))))

    def zero_hist():
        @plsc.parallel_loop(0, 1024, unroll=8)
        def _zb(j):
            hist_v[pl.ds(j * 16, 16)] = zeros16

    def bucket_count(b):
        addr = (lane << 10) + b
        return jnp.sum(plsc.load_gather(hist_v, [addr]))

    def locate(ksel):
        def tot(j):
            acc = hist_v[pl.ds(j * 16, 16)]
            for l in range(1, 16):
                acc = acc + hist_v[pl.ds(l * 1024 + j * 16, 16)]
            return acc

        def gbody(i, c):
            run, j0, above, found = c
            j = 63 - i
            s = jnp.sum(tot(j))
            nrun = run + s
            cross = jnp.logical_and(jnp.logical_not(found), nrun >= ksel)
            j0 = jnp.where(cross, j, j0)
            above = jnp.where(cross, run, above)
            return (nrun, j0, above, jnp.logical_or(found, cross))

        _, j0, above_g, _ = lax.fori_loop(
            0, 64, gbody, (jnp.int32(0), jnp.int32(0), jnp.int32(0), False))

        def bbody(i, c):
            run, b0, above, found = c
            bb = j0 * 16 + 15 - i
            s = bucket_count(bb)
            nrun = run + s
            cross = jnp.logical_and(jnp.logical_not(found), nrun >= ksel)
            b0 = jnp.where(cross, bb, b0)
            above = jnp.where(cross, run, above)
            return (nrun, b0, above, jnp.logical_or(found, cross))

        _, b0, m_above, _ = lax.fori_loop(
            0, 16, bbody, (above_g, jnp.int32(0), above_g, False))
        return b0, m_above

    def do_row(t, _):
        r = wid * ROWS_PER_W + t
        pltpu.sync_copy(logits_hbm.at[pl.ds(r * NPAD, NPAD)], row_v)

        zero_hist()

        @plsc.parallel_loop(0, NV, unroll=8)
        def _h1(i):
            key = key_of(i)
            bkt = (key >> jnp.uint32(22)).astype(jnp.int32)
            plsc.addupdate_scatter(hist_v, [(lane << 10) + bkt], ones16)
        b0, m0 = locate(jnp.int32(KSEL))
        b0_u = b0.astype(jnp.uint32)

        zero_hist()

        @plsc.parallel_loop(0, NV, unroll=8)
        def _h2(i):
            key = key_of(i)
            msk = (key >> jnp.uint32(22)) == b0_u
            bkt = ((key >> jnp.uint32(12)) & jnp.uint32(0x3FF)).astype(jnp.int32)
            plsc.addupdate_scatter(hist_v, [(lane << 10) + bkt], ones16, mask=msk)
        b1, _ = locate(jnp.int32(KSEL) - m0)
        p20_u = (b0 * 1024 + b1).astype(jnp.uint32)

        for j in range(CBUF // 16):
            cval_v[pl.ds(j * 16, 16)] = jnp.full((16,), -jnp.inf, jnp.float32)
            cidx_v[pl.ds(j * 16, 16)] = jnp.full((16,), N, jnp.int32)

        @plsc.parallel_loop(0, NV, unroll=4, carry=zeros16)
        def _cb(i, off):
            x = row_v[pl.ds(i * 16, 16)]
            u = lax.bitcast_convert_type(x, jnp.uint32)
            key = u ^ (jnp.uint32(0x80000000) | (jnp.uint32(0) - (u >> jnp.uint32(31))))
            sel = (key >> jnp.uint32(12)) >= p20_u
            inc = sel.astype(jnp.int32)
            pos = jnp.minimum(off + plsc.cumsum(inc) - 1, CBUF - 1)
            plsc.store_scatter(cidx_v, [pos], i * 16 + lane, mask=sel)
            plsc.store_scatter(cval_v, [pos], x, mask=sel)
            return off + plsc.all_reduce_population_count(sel)

        pltpu.sync_copy(cval_v.at[pl.ds(0, CAND)], cval_hbm.at[r])
        pltpu.sync_copy(cidx_v.at[pl.ds(0, CAND)], cidx_hbm.at[r])
        return 0

    lax.fori_loop(0, ROWS_PER_W, do_row, 0)


def _sc_select(logits_pad):
    mesh = plsc.VectorSubcoreMesh(core_axis_name="c", subcore_axis_name="s")
    f = pl.kernel(
        _sc_body,
        out_type=(
            jax.ShapeDtypeStruct((B, CAND), jnp.float32),
            jax.ShapeDtypeStruct((B, CAND), jnp.int32),
        ),
        mesh=mesh,
        compiler_params=pltpu.CompilerParams(needs_layout_passes=False),
        scratch_types=(
            pltpu.VMEM((NPAD,), jnp.float32),
            pltpu.VMEM((16384,), jnp.int32),
            pltpu.VMEM((CBUF,), jnp.float32),
            pltpu.VMEM((CBUF,), jnp.int32),
        ),
    )
    return f(logits_pad)


RB = 8


def _rerank_body(cval_ref, cidx_ref, cbox_ref, ts_ref,
                 scores_ref, labels_ref, boxes_ref):
    val = cval_ref[...]
    idx = cidx_ref[...]
    prob = jax.nn.sigmoid(val)
    pi = prob[:, :, None]
    pj = prob[:, None, :]
    ii = idx[:, :, None]
    ij = idx[:, None, :]
    beats = (pj > pi) | ((pj == pi) & (ij < ii))
    rank = jnp.sum(beats.astype(jnp.int32), axis=2)
    sel = rank[:, :, None] == lax.broadcasted_iota(
        jnp.int32, (RB, CAND, NUM_SELECT), 2)
    P = sel.astype(jnp.float32)

    def permute(v):
        return lax.dot_general(
            v, P, (((1,), (1,)), ((0,), (0,))),
            precision=lax.Precision.HIGHEST,
            preferred_element_type=jnp.float32)

    scores_ref[...] = permute(prob)
    lab = (idx % C).astype(jnp.float32)
    labels_ref[...] = permute(lab).astype(jnp.int32)

    qi = permute((idx // C).astype(jnp.float32)).astype(jnp.int32)
    PQ = (qi[:, :, None] == lax.broadcasted_iota(
        jnp.int32, (RB, NUM_SELECT, Q), 2)).astype(jnp.float32)

    def gatherq(v):
        return lax.dot_general(
            PQ, v, (((2,), (1,)), ((0,), (0,))),
            precision=lax.Precision.HIGHEST,
            preferred_element_type=jnp.float32)

    cx = cbox_ref[0]
    cy = cbox_ref[1]
    w = cbox_ref[2]
    h = cbox_ref[3]
    img_h = ts_ref[:, 0].astype(jnp.float32)[:, None]
    img_w = ts_ref[:, 1].astype(jnp.float32)[:, None]
    boxes_ref[0] = gatherq((cx - 0.5 * w) * img_w)
    boxes_ref[1] = gatherq((cy - 0.5 * h) * img_h)
    boxes_ref[2] = gatherq((cx + 0.5 * w) * img_w)
    boxes_ref[3] = gatherq((cy + 0.5 * h) * img_h)


def _rerank(cval, cidx, cbox, target_sizes):
    grid = (B // RB,)
    return pl.pallas_call(
        _rerank_body,
        grid=grid,
        in_specs=[
            pl.BlockSpec((RB, CAND), lambda i: (i, 0)),
            pl.BlockSpec((RB, CAND), lambda i: (i, 0)),
            pl.BlockSpec((4, RB, Q), lambda i: (0, i, 0)),
            pl.BlockSpec((RB, 2), lambda i: (i, 0)),
        ],
        out_specs=[
            pl.BlockSpec((RB, NUM_SELECT), lambda i: (i, 0)),
            pl.BlockSpec((RB, NUM_SELECT), lambda i: (i, 0)),
            pl.BlockSpec((4, RB, NUM_SELECT), lambda i: (0, i, 0)),
        ],
        out_shape=[
            jax.ShapeDtypeStruct((B, NUM_SELECT), jnp.float32),
            jax.ShapeDtypeStruct((B, NUM_SELECT), jnp.int32),
            jax.ShapeDtypeStruct((4, B, NUM_SELECT), jnp.float32),
        ],
    )(cval, cidx, cbox, target_sizes)


def kernel(outputs_pred_logits, outputs_pred_boxes, target_sizes, image_names):
    logits_flat = outputs_pred_logits.reshape(B, N)
    logits_pad = jnp.pad(logits_flat, ((0, 0), (0, NPAD - N)),
                         constant_values=float("-inf"))
    boxes_q = jnp.transpose(outputs_pred_boxes, (2, 0, 1))
    cval, cidx = _sc_select(logits_pad.reshape(B * NPAD))
    scores, labels, boxes_t = _rerank(cval, cidx, boxes_q, target_sizes)
    boxes = jnp.transpose(boxes_t, (1, 2, 0))
    return scores, labels, boxes, image_names, target_sizes


# speedup vs baseline: 1.4197x; 1.4197x over previous
import functools

import jax
import jax.numpy as jnp
from jax import lax
from jax.experimental import pallas as pl
from jax.experimental.pallas import tpu as pltpu
from jax.experimental.pallas import tpu_sc as plsc

NUM_SELECT = 300
B, Q, C = 64, 900, 91
N = Q * C
QP = 904
CP = 128
NSLAB = QP * CP
NV = NSLAB // 16
KSEL = 316
CAND = 384
CBUF = CAND + 16
BOXPAD = Q * 4 + 16
NW = 32
ROWS_PER_W = B // NW


def _sc_body(logits_hbm, cval_hbm, cidx_hbm,
             row_v, hist_v, cval_v, cidx_v):
    nc = 2
    wid = lax.axis_index("s") * nc + lax.axis_index("c")
    lane = lax.iota(jnp.int32, 16)
    lane_u = lane.astype(jnp.uint32)
    ones16 = jnp.ones((16,), jnp.int32)
    zeros16 = jnp.zeros((16,), jnp.int32)

    def key_of(i):
        x = row_v[i >> 3, pl.ds((i & 7) * 16, 16)]
        u = lax.bitcast_convert_type(x, jnp.uint32)
        return u ^ (jnp.uint32(0x80000000) | (jnp.uint32(0) - (u >> jnp.uint32(31))))

    def zero_hist():
        @plsc.parallel_loop(0, 256, unroll=8)
        def _zb(j):
            hist_v[pl.ds(j * 16, 16)] = zeros16

    def bucket_count(b):
        addr = (lane << 8) + b
        return jnp.sum(plsc.load_gather(hist_v, [addr]))

    def locate(ksel):
        def tot(j):
            acc = hist_v[pl.ds(j * 16, 16)]
            for l in range(1, 16):
                acc = acc + hist_v[pl.ds(l * 256 + j * 16, 16)]
            return acc

        def gbody(i, c):
            run, j0, above, found = c
            j = 15 - i
            s = jnp.sum(tot(j))
            nrun = run + s
            cross = jnp.logical_and(jnp.logical_not(found), nrun >= ksel)
            j0 = jnp.where(cross, j, j0)
            above = jnp.where(cross, run, above)
            return (nrun, j0, above, jnp.logical_or(found, cross))

        _, j0, above_g, _ = lax.fori_loop(
            0, 16, gbody, (jnp.int32(0), jnp.int32(0), jnp.int32(0), False))

        def bbody(i, c):
            run, b0, above, found = c
            bb = j0 * 16 + 15 - i
            s = bucket_count(bb)
            nrun = run + s
            cross = jnp.logical_and(jnp.logical_not(found), nrun >= ksel)
            b0 = jnp.where(cross, bb, b0)
            above = jnp.where(cross, run, above)
            return (nrun, b0, above, jnp.logical_or(found, cross))

        _, b0, m_above, _ = lax.fori_loop(
            0, 16, bbody, (above_g, jnp.int32(0), above_g, False))
        return b0, m_above

    def do_row(t, _):
        r = wid * ROWS_PER_W + t
        pltpu.sync_copy(logits_hbm.at[pl.ds(r * QP, QP)], row_v)

        zero_hist()

        @plsc.parallel_loop(0, NV, unroll=8)
        def _h1(i):
            key = key_of(i)
            bkt = (key >> jnp.uint32(24)).astype(jnp.int32)
            plsc.addupdate_scatter(hist_v, [(lane << 8) + bkt], ones16)
        b0, m0 = locate(jnp.int32(KSEL))
        b0_u = b0.astype(jnp.uint32)

        zero_hist()

        @plsc.parallel_loop(0, NV, unroll=8)
        def _h2(i):
            key = key_of(i)
            msk = (key >> jnp.uint32(24)) == b0_u
            bkt = ((key >> jnp.uint32(16)) & jnp.uint32(0xFF)).astype(jnp.int32)
            plsc.addupdate_scatter(hist_v, [(lane << 8) + bkt], ones16, mask=msk)
        b1, _ = locate(jnp.int32(KSEL) - m0)
        p16_u = (b0 * 256 + b1).astype(jnp.uint32)

        for j in range(CBUF // 16):
            cval_v[pl.ds(j * 16, 16)] = jnp.full((16,), -jnp.inf, jnp.float32)
            cidx_v[pl.ds(j * 16, 16)] = jnp.full((16,), NSLAB, jnp.int32)

        @plsc.parallel_loop(0, NV, unroll=4, carry=zeros16)
        def _cb(i, off):
            x = row_v[i >> 3, pl.ds((i & 7) * 16, 16)]
            u = lax.bitcast_convert_type(x, jnp.uint32)
            key = u ^ (jnp.uint32(0x80000000) | (jnp.uint32(0) - (u >> jnp.uint32(31))))
            sel = (key >> jnp.uint32(16)) >= p16_u
            inc = sel.astype(jnp.int32)
            pos = jnp.minimum(off + plsc.cumsum(inc) - 1, CBUF - 1)
            plsc.store_scatter(cidx_v, [pos], i * 16 + lane, mask=sel)
            plsc.store_scatter(cval_v, [pos], x, mask=sel)
            return off + plsc.all_reduce_population_count(sel)

        pltpu.sync_copy(cval_v.at[pl.ds(0, CAND)], cval_hbm.at[r])
        pltpu.sync_copy(cidx_v.at[pl.ds(0, CAND)], cidx_hbm.at[r])
        return 0

    lax.fori_loop(0, ROWS_PER_W, do_row, 0)


def _sc_select(logits_pad):
    mesh = plsc.VectorSubcoreMesh(core_axis_name="c", subcore_axis_name="s")
    f = pl.kernel(
        _sc_body,
        out_type=(
            jax.ShapeDtypeStruct((B, CAND), jnp.float32),
            jax.ShapeDtypeStruct((B, CAND), jnp.int32),
        ),
        mesh=mesh,
        compiler_params=pltpu.CompilerParams(needs_layout_passes=False),
        scratch_types=(
            pltpu.VMEM((QP, CP), jnp.float32),
            pltpu.VMEM((4096,), jnp.int32),
            pltpu.VMEM((CBUF,), jnp.float32),
            pltpu.VMEM((CBUF,), jnp.int32),
        ),
    )
    return f(logits_pad)


RB = 8


def _rerank_body(cval_ref, cidx_ref, cbox_ref, ts_ref,
                 scores_ref, labels_ref, boxes_ref):
    val = cval_ref[...]
    idx = cidx_ref[...]
    prob = jax.nn.sigmoid(val)
    pi = prob[:, :, None]
    pj = prob[:, None, :]
    ii = idx[:, :, None]
    ij = idx[:, None, :]
    beats = (pj > pi) | ((pj == pi) & (ij < ii))
    rank = jnp.sum(beats.astype(jnp.int32), axis=2)
    sel = rank[:, :, None] == lax.broadcasted_iota(
        jnp.int32, (RB, CAND, NUM_SELECT), 2)
    P = sel.astype(jnp.float32)

    def permute(v):
        return lax.dot_general(
            v, P, (((1,), (1,)), ((0,), (0,))),
            precision=lax.Precision.HIGHEST,
            preferred_element_type=jnp.float32)

    scores_ref[...] = permute(prob)
    lab = (idx & (CP - 1)).astype(jnp.float32)
    labels_ref[...] = permute(lab).astype(jnp.int32)

    qi = permute((idx >> 7).astype(jnp.float32)).astype(jnp.int32)
    PQ = (qi[:, :, None] == lax.broadcasted_iota(
        jnp.int32, (RB, NUM_SELECT, Q), 2)).astype(jnp.float32)

    def gatherq(v):
        return lax.dot_general(
            PQ, v, (((2,), (1,)), ((0,), (0,))),
            precision=lax.Precision.HIGHEST,
            preferred_element_type=jnp.float32)

    cx = cbox_ref[0]
    cy = cbox_ref[1]
    w = cbox_ref[2]
    h = cbox_ref[3]
    img_h = ts_ref[:, 0].astype(jnp.float32)[:, None]
    img_w = ts_ref[:, 1].astype(jnp.float32)[:, None]
    boxes_ref[0] = gatherq((cx - 0.5 * w) * img_w)
    boxes_ref[1] = gatherq((cy - 0.5 * h) * img_h)
    boxes_ref[2] = gatherq((cx + 0.5 * w) * img_w)
    boxes_ref[3] = gatherq((cy + 0.5 * h) * img_h)


def _rerank(cval, cidx, cbox, target_sizes):
    grid = (B // RB,)
    return pl.pallas_call(
        _rerank_body,
        grid=grid,
        in_specs=[
            pl.BlockSpec((RB, CAND), lambda i: (i, 0)),
            pl.BlockSpec((RB, CAND), lambda i: (i, 0)),
            pl.BlockSpec((4, RB, Q), lambda i: (0, i, 0)),
            pl.BlockSpec((RB, 2), lambda i: (i, 0)),
        ],
        out_specs=[
            pl.BlockSpec((RB, NUM_SELECT), lambda i: (i, 0)),
            pl.BlockSpec((RB, NUM_SELECT), lambda i: (i, 0)),
            pl.BlockSpec((4, RB, NUM_SELECT), lambda i: (0, i, 0)),
        ],
        out_shape=[
            jax.ShapeDtypeStruct((B, NUM_SELECT), jnp.float32),
            jax.ShapeDtypeStruct((B, NUM_SELECT), jnp.int32),
            jax.ShapeDtypeStruct((4, B, NUM_SELECT), jnp.float32),
        ],
    )(cval, cidx, cbox, target_sizes)


def kernel(outputs_pred_logits, outputs_pred_boxes, target_sizes, image_names):
    logits_pad = jnp.pad(outputs_pred_logits,
                         ((0, 0), (0, QP - Q), (0, CP - C)),
                         constant_values=float("-inf"))
    boxes_q = jnp.transpose(outputs_pred_boxes, (2, 0, 1))
    cval, cidx = _sc_select(logits_pad.reshape(B * QP, CP))
    scores, labels, boxes_t = _rerank(cval, cidx, boxes_q, target_sizes)
    boxes = jnp.transpose(boxes_t, (1, 2, 0))
    return scores, labels, boxes, image_names, target_sizes


# speedup vs baseline: 1.4202x; 1.0003x over previous
import functools

import jax
import jax.numpy as jnp
from jax import lax
from jax.experimental import pallas as pl
from jax.experimental.pallas import tpu as pltpu
from jax.experimental.pallas import tpu_sc as plsc

NUM_SELECT = 300
B, Q, C = 64, 900, 91
N = Q * C
QP = 904
CP = 128
NSLAB = QP * CP
NV = NSLAB // 16
KSEL = 316
CAND = 384
CBUF = CAND + 16
BOXPAD = Q * 4 + 16
NW = 32
ROWS_PER_W = B // NW


def _sc_body(logits_hbm, cval_hbm, cidx_hbm,
             row_v, hist_v, cval_v, cidx_v):
    nc = 2
    wid = lax.axis_index("s") * nc + lax.axis_index("c")
    lane = lax.iota(jnp.int32, 16)
    lane_u = lane.astype(jnp.uint32)
    ones16 = jnp.ones((16,), jnp.int32)
    zeros16 = jnp.zeros((16,), jnp.int32)

    def key_of(i):
        x = row_v[i >> 3, pl.ds((i & 7) * 16, 16)]
        u = lax.bitcast_convert_type(x, jnp.uint32)
        return u ^ (jnp.uint32(0x80000000) | (jnp.uint32(0) - (u >> jnp.uint32(31))))

    def zero_hist():
        @plsc.parallel_loop(0, 256, unroll=8)
        def _zb(j):
            hist_v[pl.ds(j * 16, 16)] = zeros16

    def bucket_count(b):
        addr = (lane << 8) + b
        return jnp.sum(plsc.load_gather(hist_v, [addr]))

    def locate(ksel):
        def tot(j):
            acc = hist_v[pl.ds(j * 16, 16)]
            for l in range(1, 16):
                acc = acc + hist_v[pl.ds(l * 256 + j * 16, 16)]
            return acc

        def gbody(i, c):
            run, j0, above, found = c
            j = 15 - i
            s = jnp.sum(tot(j))
            nrun = run + s
            cross = jnp.logical_and(jnp.logical_not(found), nrun >= ksel)
            j0 = jnp.where(cross, j, j0)
            above = jnp.where(cross, run, above)
            return (nrun, j0, above, jnp.logical_or(found, cross))

        _, j0, above_g, _ = lax.fori_loop(
            0, 16, gbody, (jnp.int32(0), jnp.int32(0), jnp.int32(0), False))

        def bbody(i, c):
            run, b0, above, found = c
            bb = j0 * 16 + 15 - i
            s = bucket_count(bb)
            nrun = run + s
            cross = jnp.logical_and(jnp.logical_not(found), nrun >= ksel)
            b0 = jnp.where(cross, bb, b0)
            above = jnp.where(cross, run, above)
            return (nrun, b0, above, jnp.logical_or(found, cross))

        _, b0, m_above, _ = lax.fori_loop(
            0, 16, bbody, (above_g, jnp.int32(0), above_g, False))
        return b0, m_above

    def do_row(t, _):
        r = wid * ROWS_PER_W + t
        pltpu.sync_copy(logits_hbm.at[pl.ds(r * QP, QP)], row_v)

        zero_hist()

        @plsc.parallel_loop(0, NV, unroll=8)
        def _h1(i):
            key = key_of(i)
            bkt = (key >> jnp.uint32(24)).astype(jnp.int32)
            plsc.addupdate_scatter(hist_v, [(lane << 8) + bkt], ones16)
        b0, m0 = locate(jnp.int32(KSEL))
        b0_u = b0.astype(jnp.uint32)

        zero_hist()

        @plsc.parallel_loop(0, NV, unroll=8)
        def _h2(i):
            key = key_of(i)
            msk = (key >> jnp.uint32(24)) == b0_u
            bkt = ((key >> jnp.uint32(16)) & jnp.uint32(0xFF)).astype(jnp.int32)
            plsc.addupdate_scatter(hist_v, [(lane << 8) + bkt], ones16, mask=msk)
        b1, _ = locate(jnp.int32(KSEL) - m0)
        p16_u = (b0 * 256 + b1).astype(jnp.uint32)

        for j in range(CBUF // 16):
            cval_v[pl.ds(j * 16, 16)] = jnp.full((16,), -jnp.inf, jnp.float32)
            cidx_v[pl.ds(j * 16, 16)] = jnp.full((16,), NSLAB, jnp.int32)

        @plsc.parallel_loop(0, NV, unroll=4, carry=zeros16)
        def _cb(i, off):
            x = row_v[i >> 3, pl.ds((i & 7) * 16, 16)]
            u = lax.bitcast_convert_type(x, jnp.uint32)
            key = u ^ (jnp.uint32(0x80000000) | (jnp.uint32(0) - (u >> jnp.uint32(31))))
            sel = (key >> jnp.uint32(16)) >= p16_u
            inc = sel.astype(jnp.int32)
            pos = jnp.minimum(off + plsc.cumsum(inc) - 1, CBUF - 1)
            plsc.store_scatter(cidx_v, [pos], i * 16 + lane, mask=sel)
            plsc.store_scatter(cval_v, [pos], x, mask=sel)
            return off + plsc.all_reduce_population_count(sel)

        pltpu.sync_copy(cval_v.at[pl.ds(0, CAND)], cval_hbm.at[r])
        pltpu.sync_copy(cidx_v.at[pl.ds(0, CAND)], cidx_hbm.at[r])
        return 0

    lax.fori_loop(0, ROWS_PER_W, do_row, 0)


def _sc_select(logits_pad):
    mesh = plsc.VectorSubcoreMesh(core_axis_name="c", subcore_axis_name="s")
    f = pl.kernel(
        _sc_body,
        out_type=(
            jax.ShapeDtypeStruct((B, CAND), jnp.float32),
            jax.ShapeDtypeStruct((B, CAND), jnp.int32),
        ),
        mesh=mesh,
        compiler_params=pltpu.CompilerParams(needs_layout_passes=False, use_tc_tiling_on_sc=True),
        scratch_types=(
            pltpu.VMEM((QP, CP), jnp.float32),
            pltpu.VMEM((4096,), jnp.int32),
            pltpu.VMEM((CBUF,), jnp.float32),
            pltpu.VMEM((CBUF,), jnp.int32),
        ),
    )
    return f(logits_pad)


RB = 8


def _rerank_body(cval_ref, cidx_ref, cbox_ref, ts_ref,
                 scores_ref, labels_ref, boxes_ref):
    val = cval_ref[...]
    idx = cidx_ref[...]
    prob = jax.nn.sigmoid(val)
    pi = prob[:, :, None]
    pj = prob[:, None, :]
    ii = idx[:, :, None]
    ij = idx[:, None, :]
    beats = (pj > pi) | ((pj == pi) & (ij < ii))
    rank = jnp.sum(beats.astype(jnp.int32), axis=2)
    sel = rank[:, :, None] == lax.broadcasted_iota(
        jnp.int32, (RB, CAND, NUM_SELECT), 2)
    P = sel.astype(jnp.float32)

    def permute(v):
        return lax.dot_general(
            v, P, (((1,), (1,)), ((0,), (0,))),
            precision=lax.Precision.HIGHEST,
            preferred_element_type=jnp.float32)

    scores_ref[...] = permute(prob)
    lab = (idx & (CP - 1)).astype(jnp.float32)
    labels_ref[...] = permute(lab).astype(jnp.int32)

    qi = permute((idx >> 7).astype(jnp.float32)).astype(jnp.int32)
    PQ = (qi[:, :, None] == lax.broadcasted_iota(
        jnp.int32, (RB, NUM_SELECT, Q), 2)).astype(jnp.float32)

    def gatherq(v):
        return lax.dot_general(
            PQ, v, (((2,), (1,)), ((0,), (0,))),
            precision=lax.Precision.HIGHEST,
            preferred_element_type=jnp.float32)

    cx = cbox_ref[0]
    cy = cbox_ref[1]
    w = cbox_ref[2]
    h = cbox_ref[3]
    img_h = ts_ref[:, 0].astype(jnp.float32)[:, None]
    img_w = ts_ref[:, 1].astype(jnp.float32)[:, None]
    boxes_ref[0] = gatherq((cx - 0.5 * w) * img_w)
    boxes_ref[1] = gatherq((cy - 0.5 * h) * img_h)
    boxes_ref[2] = gatherq((cx + 0.5 * w) * img_w)
    boxes_ref[3] = gatherq((cy + 0.5 * h) * img_h)


def _rerank(cval, cidx, cbox, target_sizes):
    grid = (B // RB,)
    return pl.pallas_call(
        _rerank_body,
        grid=grid,
        in_specs=[
            pl.BlockSpec((RB, CAND), lambda i: (i, 0)),
            pl.BlockSpec((RB, CAND), lambda i: (i, 0)),
            pl.BlockSpec((4, RB, Q), lambda i: (0, i, 0)),
            pl.BlockSpec((RB, 2), lambda i: (i, 0)),
        ],
        out_specs=[
            pl.BlockSpec((RB, NUM_SELECT), lambda i: (i, 0)),
            pl.BlockSpec((RB, NUM_SELECT), lambda i: (i, 0)),
            pl.BlockSpec((4, RB, NUM_SELECT), lambda i: (0, i, 0)),
        ],
        out_shape=[
            jax.ShapeDtypeStruct((B, NUM_SELECT), jnp.float32),
            jax.ShapeDtypeStruct((B, NUM_SELECT), jnp.int32),
            jax.ShapeDtypeStruct((4, B, NUM_SELECT), jnp.float32),
        ],
    )(cval, cidx, cbox, target_sizes)


def kernel(outputs_pred_logits, outputs_pred_boxes, target_sizes, image_names):
    logits_pad = jnp.pad(outputs_pred_logits,
                         ((0, 0), (0, QP - Q), (0, CP - C)),
                         constant_values=float("-inf"))
    boxes_q = jnp.transpose(outputs_pred_boxes, (2, 0, 1))
    cval, cidx = _sc_select(logits_pad.reshape(B * QP, CP))
    scores, labels, boxes_t = _rerank(cval, cidx, boxes_q, target_sizes)
    boxes = jnp.transpose(boxes_t, (1, 2, 0))
    return scores, labels, boxes, image_names, target_sizes


# speedup vs baseline: 1.6807x; 1.1834x over previous
import functools

import jax
import jax.numpy as jnp
from jax import lax
from jax.experimental import pallas as pl
from jax.experimental.pallas import tpu as pltpu
from jax.experimental.pallas import tpu_sc as plsc

NUM_SELECT = 300
B, Q, C = 64, 900, 91
N = Q * C
QP = 904
CP = 128
NSLAB = QP * CP
NV = NSLAB // 16
KSEL = 316
CAND = 384
CBUF = CAND + 16
BOXPAD = Q * 4 + 16
NW = 32
ROWS_PER_W = B // NW


def _sc_body(logits_hbm, cval_hbm, cidx_hbm,
             row_v, hist_v, cval_v, cidx_v):
    nc = 2
    wid = lax.axis_index("s") * nc + lax.axis_index("c")
    lane = lax.iota(jnp.int32, 16)
    lane_u = lane.astype(jnp.uint32)
    ones16 = jnp.ones((16,), jnp.int32)
    zeros16 = jnp.zeros((16,), jnp.int32)

    def key_of(i):
        x = row_v[i >> 3, pl.ds((i & 7) * 16, 16)]
        u = lax.bitcast_convert_type(x, jnp.uint32)
        return u ^ (jnp.uint32(0x80000000) | (jnp.uint32(0) - (u >> jnp.uint32(31))))

    def zero_hist():
        @plsc.parallel_loop(0, 256, unroll=8)
        def _zb(j):
            hist_v[pl.ds(j * 16, 16)] = zeros16

    def bucket_count(b):
        return jnp.sum(hist_v[pl.ds(b * 16, 16)])

    def locate(ksel):
        def tot(j):
            acc = hist_v[pl.ds(j * 256, 16)]
            for l in range(1, 16):
                acc = acc + hist_v[pl.ds(j * 256 + l * 16, 16)]
            return acc

        def gbody(i, c):
            run, j0, above, found = c
            j = 15 - i
            s = jnp.sum(tot(j))
            nrun = run + s
            cross = jnp.logical_and(jnp.logical_not(found), nrun >= ksel)
            j0 = jnp.where(cross, j, j0)
            above = jnp.where(cross, run, above)
            return (nrun, j0, above, jnp.logical_or(found, cross))

        _, j0, above_g, _ = lax.fori_loop(
            0, 16, gbody, (jnp.int32(0), jnp.int32(0), jnp.int32(0), False))

        def bbody(i, c):
            run, b0, above, found = c
            bb = j0 * 16 + 15 - i
            s = bucket_count(bb)
            nrun = run + s
            cross = jnp.logical_and(jnp.logical_not(found), nrun >= ksel)
            b0 = jnp.where(cross, bb, b0)
            above = jnp.where(cross, run, above)
            return (nrun, b0, above, jnp.logical_or(found, cross))

        _, b0, m_above, _ = lax.fori_loop(
            0, 16, bbody, (above_g, jnp.int32(0), above_g, False))
        return b0, m_above

    def do_row(t, _):
        r = wid * ROWS_PER_W + t
        pltpu.sync_copy(logits_hbm.at[pl.ds(r * QP, QP)], row_v)

        zero_hist()

        @plsc.parallel_loop(0, NV, unroll=8)
        def _h1(i):
            key = key_of(i)
            bkt = (key >> jnp.uint32(24)).astype(jnp.int32)
            plsc.addupdate_scatter(hist_v, [(bkt << 4) + lane], ones16)
        b0, m0 = locate(jnp.int32(KSEL))
        b0_u = b0.astype(jnp.uint32)

        zero_hist()

        @plsc.parallel_loop(0, NV, unroll=8)
        def _h2(i):
            key = key_of(i)
            msk = (key >> jnp.uint32(24)) == b0_u
            bkt = ((key >> jnp.uint32(16)) & jnp.uint32(0xFF)).astype(jnp.int32)
            plsc.addupdate_scatter(hist_v, [(bkt << 4) + lane], ones16, mask=msk)
        b1, _ = locate(jnp.int32(KSEL) - m0)
        p16_u = (b0 * 256 + b1).astype(jnp.uint32)

        for j in range(CBUF // 16):
            cval_v[pl.ds(j * 16, 16)] = jnp.full((16,), -jnp.inf, jnp.float32)
            cidx_v[pl.ds(j * 16, 16)] = jnp.full((16,), NSLAB, jnp.int32)

        @plsc.parallel_loop(0, NV, unroll=4, carry=zeros16)
        def _cb(i, off):
            x = row_v[i >> 3, pl.ds((i & 7) * 16, 16)]
            u = lax.bitcast_convert_type(x, jnp.uint32)
            key = u ^ (jnp.uint32(0x80000000) | (jnp.uint32(0) - (u >> jnp.uint32(31))))
            sel = (key >> jnp.uint32(16)) >= p16_u
            inc = sel.astype(jnp.int32)
            pos = jnp.minimum(off + plsc.cumsum(inc) - 1, CBUF - 1)
            plsc.store_scatter(cidx_v, [pos], i * 16 + lane, mask=sel)
            plsc.store_scatter(cval_v, [pos], x, mask=sel)
            return off + plsc.all_reduce_population_count(sel)

        pltpu.sync_copy(cval_v.at[pl.ds(0, CAND)], cval_hbm.at[r])
        pltpu.sync_copy(cidx_v.at[pl.ds(0, CAND)], cidx_hbm.at[r])
        return 0

    lax.fori_loop(0, ROWS_PER_W, do_row, 0)


def _sc_select(logits_pad):
    mesh = plsc.VectorSubcoreMesh(core_axis_name="c", subcore_axis_name="s")
    f = pl.kernel(
        _sc_body,
        out_type=(
            jax.ShapeDtypeStruct((B, CAND), jnp.float32),
            jax.ShapeDtypeStruct((B, CAND), jnp.int32),
        ),
        mesh=mesh,
        compiler_params=pltpu.CompilerParams(needs_layout_passes=False),
        scratch_types=(
            pltpu.VMEM((QP, CP), jnp.float32),
            pltpu.VMEM((4096,), jnp.int32),
            pltpu.VMEM((CBUF,), jnp.float32),
            pltpu.VMEM((CBUF,), jnp.int32),
        ),
    )
    return f(logits_pad)


RB = 8


def _rerank_body(cval_ref, cidx_ref, cbox_ref, ts_ref,
                 scores_ref, labels_ref, boxes_ref):
    val = cval_ref[...]
    idx = cidx_ref[...]
    prob = jax.nn.sigmoid(val)
    pi = prob[:, :, None]
    pj = prob[:, None, :]
    ii = idx[:, :, None]
    ij = idx[:, None, :]
    beats = (pj > pi) | ((pj == pi) & (ij < ii))
    rank = jnp.sum(beats.astype(jnp.int32), axis=2)
    sel = rank[:, :, None] == lax.broadcasted_iota(
        jnp.int32, (RB, CAND, NUM_SELECT), 2)
    P = sel.astype(jnp.float32)

    def permute(v):
        return lax.dot_general(
            v, P, (((1,), (1,)), ((0,), (0,))),
            precision=lax.Precision.HIGHEST,
            preferred_element_type=jnp.float32)

    scores_ref[...] = permute(prob)
    lab = (idx & (CP - 1)).astype(jnp.float32)
    labels_ref[...] = permute(lab).astype(jnp.int32)

    qi = permute((idx >> 7).astype(jnp.float32)).astype(jnp.int32)
    PQ = (qi[:, :, None] == lax.broadcasted_iota(
        jnp.int32, (RB, NUM_SELECT, Q), 2)).astype(jnp.float32)

    def gatherq(v):
        return lax.dot_general(
            PQ, v, (((2,), (1,)), ((0,), (0,))),
            precision=lax.Precision.HIGHEST,
            preferred_element_type=jnp.float32)

    cx = cbox_ref[0]
    cy = cbox_ref[1]
    w = cbox_ref[2]
    h = cbox_ref[3]
    img_h = ts_ref[:, 0].astype(jnp.float32)[:, None]
    img_w = ts_ref[:, 1].astype(jnp.float32)[:, None]
    boxes_ref[0] = gatherq((cx - 0.5 * w) * img_w)
    boxes_ref[1] = gatherq((cy - 0.5 * h) * img_h)
    boxes_ref[2] = gatherq((cx + 0.5 * w) * img_w)
    boxes_ref[3] = gatherq((cy + 0.5 * h) * img_h)


def _rerank(cval, cidx, cbox, target_sizes):
    grid = (B // RB,)
    return pl.pallas_call(
        _rerank_body,
        grid=grid,
        in_specs=[
            pl.BlockSpec((RB, CAND), lambda i: (i, 0)),
            pl.BlockSpec((RB, CAND), lambda i: (i, 0)),
            pl.BlockSpec((4, RB, Q), lambda i: (0, i, 0)),
            pl.BlockSpec((RB, 2), lambda i: (i, 0)),
        ],
        out_specs=[
            pl.BlockSpec((RB, NUM_SELECT), lambda i: (i, 0)),
            pl.BlockSpec((RB, NUM_SELECT), lambda i: (i, 0)),
            pl.BlockSpec((4, RB, NUM_SELECT), lambda i: (0, i, 0)),
        ],
        out_shape=[
            jax.ShapeDtypeStruct((B, NUM_SELECT), jnp.float32),
            jax.ShapeDtypeStruct((B, NUM_SELECT), jnp.int32),
            jax.ShapeDtypeStruct((4, B, NUM_SELECT), jnp.float32),
        ],
    )(cval, cidx, cbox, target_sizes)


def kernel(outputs_pred_logits, outputs_pred_boxes, target_sizes, image_names):
    logits_pad = jnp.pad(outputs_pred_logits,
                         ((0, 0), (0, QP - Q), (0, CP - C)),
                         constant_values=float("-inf"))
    boxes_q = jnp.transpose(outputs_pred_boxes, (2, 0, 1))
    cval, cidx = _sc_select(logits_pad.reshape(B * QP, CP))
    scores, labels, boxes_t = _rerank(cval, cidx, boxes_q, target_sizes)
    boxes = jnp.transpose(boxes_t, (1, 2, 0))
    return scores, labels, boxes, image_names, target_sizes


# speedup vs baseline: 1.6881x; 1.0044x over previous
import functools

import jax
import jax.numpy as jnp
from jax import lax
from jax.experimental import pallas as pl
from jax.experimental.pallas import tpu as pltpu
from jax.experimental.pallas import tpu_sc as plsc

NUM_SELECT = 300
B, Q, C = 64, 900, 91
N = Q * C
QP = 904
CP = 128
NSLAB = QP * CP
NV = NSLAB // 16
KSEL = 316
CAND = 384
CBUF = CAND + 16
BOXPAD = Q * 4 + 16
NW = 32
ROWS_PER_W = B // NW


def _sc_body(logits_hbm, cval_hbm, cidx_hbm,
             row_v, hist_v, cval_v, cidx_v):
    nc = 2
    wid = lax.axis_index("s") * nc + lax.axis_index("c")
    lane = lax.iota(jnp.int32, 16)
    lane_u = lane.astype(jnp.uint32)
    ones16 = jnp.ones((16,), jnp.int32)
    zeros16 = jnp.zeros((16,), jnp.int32)

    def key_of(i):
        x = row_v[i >> 3, pl.ds((i & 7) * 16, 16)]
        u = lax.bitcast_convert_type(x, jnp.uint32)
        return u ^ (jnp.uint32(0x80000000) | (jnp.uint32(0) - (u >> jnp.uint32(31))))

    def zero_hist():
        @plsc.parallel_loop(0, 256, unroll=8)
        def _zb(j):
            hist_v[pl.ds(j * 16, 16)] = zeros16

    def bucket_count(b):
        return jnp.sum(hist_v[pl.ds(b * 16, 16)])

    def locate(ksel):
        def tot(j):
            acc = hist_v[pl.ds(j * 256, 16)]
            for l in range(1, 16):
                acc = acc + hist_v[pl.ds(j * 256 + l * 16, 16)]
            return acc

        def gbody(i, c):
            run, j0, above, found = c
            j = 15 - i
            s = jnp.sum(tot(j))
            nrun = run + s
            cross = jnp.logical_and(jnp.logical_not(found), nrun >= ksel)
            j0 = jnp.where(cross, j, j0)
            above = jnp.where(cross, run, above)
            return (nrun, j0, above, jnp.logical_or(found, cross))

        _, j0, above_g, _ = lax.fori_loop(
            0, 16, gbody, (jnp.int32(0), jnp.int32(0), jnp.int32(0), False))

        def bbody(i, c):
            run, b0, above, found = c
            bb = j0 * 16 + 15 - i
            s = bucket_count(bb)
            nrun = run + s
            cross = jnp.logical_and(jnp.logical_not(found), nrun >= ksel)
            b0 = jnp.where(cross, bb, b0)
            above = jnp.where(cross, run, above)
            return (nrun, b0, above, jnp.logical_or(found, cross))

        _, b0, m_above, _ = lax.fori_loop(
            0, 16, bbody, (above_g, jnp.int32(0), above_g, False))
        return b0, m_above

    def do_row(t, _):
        r = wid * ROWS_PER_W + t
        pltpu.sync_copy(logits_hbm.at[pl.ds(r * QP, QP)], row_v)

        zero_hist()

        @plsc.parallel_loop(0, NV, unroll=8)
        def _h1(i):
            key = key_of(i)
            bkt = (key >> jnp.uint32(24)).astype(jnp.int32)
            plsc.addupdate_scatter(hist_v, [(bkt << 4) + lane], ones16)
        b0, m0 = locate(jnp.int32(KSEL))
        b0_u = b0.astype(jnp.uint32)

        zero_hist()

        @plsc.parallel_loop(0, NV, unroll=8)
        def _h2(i):
            key = key_of(i)
            msk = (key >> jnp.uint32(24)) == b0_u
            bkt = ((key >> jnp.uint32(16)) & jnp.uint32(0xFF)).astype(jnp.int32)
            plsc.addupdate_scatter(hist_v, [(bkt << 4) + lane], ones16, mask=msk)
        b1, _ = locate(jnp.int32(KSEL) - m0)
        p16_u = (b0 * 256 + b1).astype(jnp.uint32)

        for j in range(CBUF // 16):
            cval_v[pl.ds(j * 16, 16)] = jnp.full((16,), -jnp.inf, jnp.float32)
            cidx_v[pl.ds(j * 16, 16)] = jnp.full((16,), NSLAB, jnp.int32)

        @plsc.parallel_loop(0, NV, unroll=4, carry=zeros16)
        def _cb(i, off):
            x = row_v[i >> 3, pl.ds((i & 7) * 16, 16)]
            u = lax.bitcast_convert_type(x, jnp.uint32)
            key = u ^ (jnp.uint32(0x80000000) | (jnp.uint32(0) - (u >> jnp.uint32(31))))
            sel = (key >> jnp.uint32(16)) >= p16_u
            inc = sel.astype(jnp.int32)
            pos = jnp.minimum(off + plsc.cumsum(inc) - 1, CBUF - 1)
            plsc.store_scatter(cidx_v, [pos], i * 16 + lane, mask=sel)
            plsc.store_scatter(cval_v, [pos], x, mask=sel)
            return off + plsc.all_reduce_population_count(sel)

        pltpu.sync_copy(cval_v.at[pl.ds(0, CAND)], cval_hbm.at[r])
        pltpu.sync_copy(cidx_v.at[pl.ds(0, CAND)], cidx_hbm.at[r])
        return 0

    lax.fori_loop(0, ROWS_PER_W, do_row, 0)


def _sc_select(logits_pad):
    mesh = plsc.VectorSubcoreMesh(core_axis_name="c", subcore_axis_name="s")
    f = pl.kernel(
        _sc_body,
        out_type=(
            jax.ShapeDtypeStruct((B, CAND), jnp.float32),
            jax.ShapeDtypeStruct((B, CAND), jnp.int32),
        ),
        mesh=mesh,
        compiler_params=pltpu.CompilerParams(needs_layout_passes=False),
        scratch_types=(
            pltpu.VMEM((QP, CP), jnp.float32),
            pltpu.VMEM((4096,), jnp.int32),
            pltpu.VMEM((CBUF,), jnp.float32),
            pltpu.VMEM((CBUF,), jnp.int32),
        ),
    )
    return f(logits_pad)


RB = 16


def _rerank_body(cval_ref, cidx_ref, cbox_ref, ts_ref,
                 scores_ref, labels_ref, boxes_ref):
    val = cval_ref[...]
    idx = cidx_ref[...]
    prob = jax.nn.sigmoid(val)
    pi = prob[:, :, None]
    pj = prob[:, None, :]
    ii = idx[:, :, None]
    ij = idx[:, None, :]
    beats = (pj > pi) | ((pj == pi) & (ij < ii))
    rank = jnp.sum(beats.astype(jnp.int32), axis=2)
    sel = rank[:, :, None] == lax.broadcasted_iota(
        jnp.int32, (RB, CAND, NUM_SELECT), 2)
    P = sel.astype(jnp.float32)

    def permute(v):
        return lax.dot_general(
            v, P, (((1,), (1,)), ((0,), (0,))),
            precision=lax.Precision.HIGHEST,
            preferred_element_type=jnp.float32)

    scores_ref[...] = permute(prob)
    lab = (idx & (CP - 1)).astype(jnp.float32)
    labels_ref[...] = permute(lab).astype(jnp.int32)

    qi = permute((idx >> 7).astype(jnp.float32)).astype(jnp.int32)
    PQ = (qi[:, :, None] == lax.broadcasted_iota(
        jnp.int32, (RB, NUM_SELECT, Q), 2)).astype(jnp.float32)

    def gatherq(v):
        return lax.dot_general(
            PQ, v, (((2,), (1,)), ((0,), (0,))),
            precision=lax.Precision.HIGHEST,
            preferred_element_type=jnp.float32)

    cx = cbox_ref[0]
    cy = cbox_ref[1]
    w = cbox_ref[2]
    h = cbox_ref[3]
    img_h = ts_ref[:, 0].astype(jnp.float32)[:, None]
    img_w = ts_ref[:, 1].astype(jnp.float32)[:, None]
    boxes_ref[0] = gatherq((cx - 0.5 * w) * img_w)
    boxes_ref[1] = gatherq((cy - 0.5 * h) * img_h)
    boxes_ref[2] = gatherq((cx + 0.5 * w) * img_w)
    boxes_ref[3] = gatherq((cy + 0.5 * h) * img_h)


def _rerank(cval, cidx, cbox, target_sizes):
    grid = (B // RB,)
    return pl.pallas_call(
        _rerank_body,
        grid=grid,
        in_specs=[
            pl.BlockSpec((RB, CAND), lambda i: (i, 0)),
            pl.BlockSpec((RB, CAND), lambda i: (i, 0)),
            pl.BlockSpec((4, RB, Q), lambda i: (0, i, 0)),
            pl.BlockSpec((RB, 2), lambda i: (i, 0)),
        ],
        out_specs=[
            pl.BlockSpec((RB, NUM_SELECT), lambda i: (i, 0)),
            pl.BlockSpec((RB, NUM_SELECT), lambda i: (i, 0)),
            pl.BlockSpec((4, RB, NUM_SELECT), lambda i: (0, i, 0)),
        ],
        out_shape=[
            jax.ShapeDtypeStruct((B, NUM_SELECT), jnp.float32),
            jax.ShapeDtypeStruct((B, NUM_SELECT), jnp.int32),
            jax.ShapeDtypeStruct((4, B, NUM_SELECT), jnp.float32),
        ],
    )(cval, cidx, cbox, target_sizes)


def kernel(outputs_pred_logits, outputs_pred_boxes, target_sizes, image_names):
    logits_pad = jnp.pad(outputs_pred_logits,
                         ((0, 0), (0, QP - Q), (0, CP - C)),
                         constant_values=float("-inf"))
    boxes_q = jnp.transpose(outputs_pred_boxes, (2, 0, 1))
    cval, cidx = _sc_select(logits_pad.reshape(B * QP, CP))
    scores, labels, boxes_t = _rerank(cval, cidx, boxes_q, target_sizes)
    boxes = jnp.transpose(boxes_t, (1, 2, 0))
    return scores, labels, boxes, image_names, target_sizes
